# baseline (speedup 1.0000x reference)
import functools

import numpy as np
import jax
import jax.numpy as jnp
from jax import lax
from jax.experimental import pallas as pl
from jax.experimental.pallas import tpu as pltpu
from jax.experimental.pallas import tpu_sc as plsc

_OUT_SHAPE = (512, 512)
_NUM_CORES = 2
_NUM_SUBCORES = 16
_NUM_WORKERS = _NUM_CORES * _NUM_SUBCORES
_Q = 64
_NBUF = 4


def _polar_grid_state(B, H, W):
    cy, cx = (H - 1) / 2.0, (W - 1) / 2.0
    max_radius = min(cy, cx)
    n_r, n_t = _OUT_SHAPE
    radii = np.linspace(0.0, max_radius, n_r)
    angles = np.linspace(0.0, 2.0 * np.pi, n_t, endpoint=False)
    rr, tt = np.meshgrid(radii, angles, indexing="ij")
    ys = (cy + rr * np.sin(tt)).astype(np.float32).reshape(-1)
    xs = (cx + rr * np.cos(tt)).astype(np.float32).reshape(-1)
    fy = np.clip(np.floor(ys), 0, H - 2).astype(np.int32)
    fx = np.clip(np.floor(xs), 0, W - 2).astype(np.int32)
    ay = np.clip(ys - fy.astype(np.float32), 0.0, 1.0).astype(np.float32)
    ax = np.clip(xs - fx.astype(np.float32), 0.0, 1.0).astype(np.float32)
    base = fy * np.int32(W) + fx
    n_pts = n_r * n_t
    N = B * n_pts
    rows = (np.arange(B, dtype=np.int32)[:, None] * np.int32(H * W)
            + base[None, :]).reshape(N)
    corners = np.stack([rows, rows + 1, rows + W, rows + W + 1], axis=-1)
    weights = np.tile(np.stack([ax, ay], axis=-1), (B, 1))
    nchunks = N // _Q
    idx_all = np.ascontiguousarray(
        corners.reshape(nchunks, _Q, 4).transpose(0, 2, 1)
    ).reshape(nchunks, 2, 2 * _Q)
    w_all = np.ascontiguousarray(
        weights.reshape(nchunks, _Q, 2).transpose(0, 2, 1))
    return idx_all, w_all


@functools.lru_cache(maxsize=None)
def _build(B, H, W, C):
    n_r, n_t = _OUT_SHAPE
    N = B * n_r * n_t
    assert N % (_NUM_WORKERS * _Q) == 0 and C % 16 == 0
    nchunks = N // (_NUM_WORKERS * _Q)
    assert nchunks % _NBUF == 0 and nchunks >= 2 * _NBUF
    idx_all, w_all = _polar_grid_state(B, H, W)

    mesh = plsc.VectorSubcoreMesh(core_axis_name="c", subcore_axis_name="s")

    @functools.partial(
        pl.kernel,
        mesh=mesh,
        out_type=jax.ShapeDtypeStruct((B, n_r, n_t, C), jnp.float32),
        scratch_types=[
            pltpu.VMEM((_NBUF, 2, 2 * _Q), jnp.int32),
            pltpu.VMEM((_NBUF, 2, _Q), jnp.float32),
            pltpu.VMEM((_NBUF, 2, 2 * _Q, C), jnp.float32),
            pltpu.VMEM((2, _Q, C), jnp.float32),
        ] + [pltpu.SemaphoreType.DMA] * (3 * _NBUF + 2),
        compiler_params=pltpu.CompilerParams(use_tc_tiling_on_sc=False),
    )
    def polar_sc(table, idx_hbm, w_hbm, out_hbm, idx_v, w_v, rows_v,
                 out_v, *sems):
        sem_g = sems[0:_NBUF]
        sem_i = sems[_NBUF:2 * _NBUF]
        sem_w = sems[2 * _NBUF:3 * _NBUF]
        sem_o = sems[3 * _NBUF:3 * _NBUF + 2]
        wid = lax.axis_index("s") * _NUM_CORES + lax.axis_index("c")
        c0 = wid * nchunks

        def gather_descr(b):
            return [
                pltpu.make_async_copy(table.at[idx_v.at[b, k]],
                                      rows_v.at[b, k], sem_g[b])
                for k in range(2)
            ]

        def idx_descr(c, b):
            return pltpu.make_async_copy(idx_hbm.at[c0 + c], idx_v.at[b],
                                         sem_i[b])

        def w_descr(c, b):
            return pltpu.make_async_copy(w_hbm.at[c0 + c], w_v.at[b],
                                         sem_w[b])

        chunks_per_row = n_t // _Q

        def out_descr(c, ob):
            g = c0 + c
            row = g // chunks_per_row
            x0 = (g % chunks_per_row) * _Q
            return pltpu.make_async_copy(
                out_v.at[ob],
                out_hbm.at[row // n_r, row % n_r, pl.ds(x0, _Q)],
                sem_o[ob])

        def compute(b, ob):
            def q_body(qg, carry2):
                q0 = qg * 16
                axv = w_v[b, 0, pl.ds(q0, 16)]
                ayv = w_v[b, 1, pl.ds(q0, 16)]
                for j in range(16):
                    q = q0 + j
                    ax = axv[j]
                    ay = ayv[j]
                    for cb in range(C // 16):
                        sl = pl.ds(cb * 16, 16)
                        tl = rows_v[b, 0, q, sl]
                        tr = rows_v[b, 0, _Q + q, sl]
                        bl = rows_v[b, 1, q, sl]
                        br = rows_v[b, 1, _Q + q, sl]
                        top = tl + ax * (tr - tl)
                        bot = bl + ax * (br - bl)
                        out_v[ob, q, sl] = top + ay * (bot - top)
                return carry2

            lax.fori_loop(0, _Q // 16, q_body, 0)

        for c in range(_NBUF - 1):
            idx_descr(c, c).start()
            idx_descr(c, c).wait()
            w_descr(c, c).start()
            w_descr(c, c).wait()
            for d in gather_descr(c):
                d.start()
        idx_descr(_NBUF - 1, _NBUF - 1).start()
        w_descr(_NBUF - 1, _NBUF - 1).start()

        def ring_body(i, carry):
            for b in range(_NBUF):
                c = _NBUF * i + b
                ob = b % 2
                nxt = (b + _NBUF - 1) % _NBUF

                for d in gather_descr(b):
                    d.wait()

                @pl.when(c + _NBUF < nchunks)
                def _():
                    idx_descr(c + _NBUF, b).start()

                @pl.when(c + _NBUF - 1 < nchunks)
                def _():
                    idx_descr(c + _NBUF - 1, nxt).wait()
                    w_descr(c + _NBUF - 1, nxt).wait()
                    for d in gather_descr(nxt):
                        d.start()

                @pl.when(c >= 2)
                def _():
                    out_descr(c - 2, ob).wait()

                compute(b, ob)
                out_descr(c, ob).start()

                @pl.when(c + _NBUF < nchunks)
                def _():
                    w_descr(c + _NBUF, b).start()
            return carry

        lax.fori_loop(0, nchunks // _NBUF, ring_body, 0)

        out_descr(0, 0).wait()
        out_descr(0, 1).wait()

    return polar_sc, idx_all, w_all


def kernel(img):
    B, H, W, C = img.shape
    polar_sc, idx_all, w_all = _build(B, H, W, C)
    table = img.reshape(B * H * W, C)
    return polar_sc(table, jnp.asarray(idx_all), jnp.asarray(w_all))


# speedup vs baseline: 1.0405x; 1.0405x over previous
import functools

import numpy as np
import jax
import jax.numpy as jnp
from jax import lax
from jax.experimental import pallas as pl
from jax.experimental.pallas import tpu as pltpu
from jax.experimental.pallas import tpu_sc as plsc

_OUT_SHAPE = (512, 512)
_NUM_CORES = 2
_NUM_SUBCORES = 16
_NUM_WORKERS = _NUM_CORES * _NUM_SUBCORES
_Q = 64
_NBUF = 4


def _polar_grid_state(B, H, W):
    cy, cx = (H - 1) / 2.0, (W - 1) / 2.0
    max_radius = min(cy, cx)
    n_r, n_t = _OUT_SHAPE
    radii = np.linspace(0.0, max_radius, n_r)
    angles = np.linspace(0.0, 2.0 * np.pi, n_t, endpoint=False)
    rr, tt = np.meshgrid(radii, angles, indexing="ij")
    ys = (cy + rr * np.sin(tt)).astype(np.float32).reshape(-1)
    xs = (cx + rr * np.cos(tt)).astype(np.float32).reshape(-1)
    fy = np.clip(np.floor(ys), 0, H - 2).astype(np.int32)
    fx = np.clip(np.floor(xs), 0, W - 2).astype(np.int32)
    ay = np.clip(ys - fy.astype(np.float32), 0.0, 1.0).astype(np.float32)
    ax = np.clip(xs - fx.astype(np.float32), 0.0, 1.0).astype(np.float32)
    base = fy * np.int32(W) + fx
    n_pts = n_r * n_t
    N = B * n_pts
    rows = (np.arange(B, dtype=np.int32)[:, None] * np.int32(H * W)
            + base[None, :]).reshape(N)
    corners = np.stack([rows, rows + 1, rows + W, rows + W + 1], axis=-1)
    weights = np.tile(np.stack([ax, ay], axis=-1), (B, 1))
    nchunks = N // _Q
    idx_all = np.ascontiguousarray(
        corners.reshape(nchunks, _Q, 4).transpose(0, 2, 1))
    w_all = np.ascontiguousarray(
        weights.reshape(nchunks, _Q, 2).transpose(0, 2, 1))
    return idx_all, w_all


@functools.lru_cache(maxsize=None)
def _build(B, H, W, C):
    n_r, n_t = _OUT_SHAPE
    N = B * n_r * n_t
    assert N % (_NUM_WORKERS * _Q) == 0 and C % 16 == 0
    nchunks = N // (_NUM_WORKERS * _Q)
    assert nchunks % _NBUF == 0 and nchunks >= 2 * _NBUF
    idx_all, w_all = _polar_grid_state(B, H, W)

    mesh = plsc.VectorSubcoreMesh(core_axis_name="c", subcore_axis_name="s")

    @functools.partial(
        pl.kernel,
        mesh=mesh,
        out_type=jax.ShapeDtypeStruct((B, n_r, n_t, C), jnp.float32),
        scratch_types=[
            pltpu.VMEM((_NBUF, 4, _Q), jnp.int32),
            pltpu.VMEM((_NBUF, 2, _Q), jnp.float32),
            pltpu.VMEM((_NBUF, 4, _Q, C), jnp.float32),
            pltpu.VMEM((2, _Q, C), jnp.float32),
        ] + [pltpu.SemaphoreType.DMA] * (3 * _NBUF + 2),
        compiler_params=pltpu.CompilerParams(use_tc_tiling_on_sc=False),
    )
    def polar_sc(table, idx_hbm, w_hbm, out_hbm, idx_v, w_v, rows_v,
                 out_v, *sems):
        sem_g = sems[0:_NBUF]
        sem_i = sems[_NBUF:2 * _NBUF]
        sem_w = sems[2 * _NBUF:3 * _NBUF]
        sem_o = sems[3 * _NBUF:3 * _NBUF + 2]
        wid = lax.axis_index("s") * _NUM_CORES + lax.axis_index("c")
        c0 = wid * nchunks

        def gather_descr(b):
            return [
                pltpu.make_async_copy(
                    table.at[idx_v.at[b, k, pl.ds(h * (_Q // 2), _Q // 2)]],
                    rows_v.at[b, k, pl.ds(h * (_Q // 2), _Q // 2)],
                    sem_g[b])
                for k in range(4)
                for h in range(2)
            ]

        def idx_descr(c, b):
            return pltpu.make_async_copy(idx_hbm.at[c0 + c], idx_v.at[b],
                                         sem_i[b])

        def w_descr(c, b):
            return pltpu.make_async_copy(w_hbm.at[c0 + c], w_v.at[b],
                                         sem_w[b])

        chunks_per_row = n_t // _Q

        def out_descr(c, ob):
            g = c0 + c
            row = g // chunks_per_row
            x0 = (g % chunks_per_row) * _Q
            return pltpu.make_async_copy(
                out_v.at[ob],
                out_hbm.at[row // n_r, row % n_r, pl.ds(x0, _Q)],
                sem_o[ob])

        def compute(b, ob):
            def q_body(qg, carry2):
                q0 = qg * 16
                axv = w_v[b, 0, pl.ds(q0, 16)]
                ayv = w_v[b, 1, pl.ds(q0, 16)]
                for j in range(16):
                    q = q0 + j
                    ax = axv[j]
                    ay = ayv[j]
                    for cb in range(C // 16):
                        sl = pl.ds(cb * 16, 16)
                        tl = rows_v[b, 0, q, sl]
                        tr = rows_v[b, 1, q, sl]
                        bl = rows_v[b, 2, q, sl]
                        br = rows_v[b, 3, q, sl]
                        top = tl + ax * (tr - tl)
                        bot = bl + ax * (br - bl)
                        out_v[ob, q, sl] = top + ay * (bot - top)
                return carry2

            lax.fori_loop(0, _Q // 16, q_body, 0)

        for c in range(_NBUF - 1):
            idx_descr(c, c).start()
            idx_descr(c, c).wait()
            w_descr(c, c).start()
            w_descr(c, c).wait()
            for d in gather_descr(c):
                d.start()
        idx_descr(_NBUF - 1, _NBUF - 1).start()
        w_descr(_NBUF - 1, _NBUF - 1).start()

        def ring_body(i, carry):
            for b in range(_NBUF):
                c = _NBUF * i + b
                ob = b % 2
                nxt = (b + _NBUF - 1) % _NBUF

                for d in gather_descr(b):
                    d.wait()

                @pl.when(c + _NBUF < nchunks)
                def _():
                    idx_descr(c + _NBUF, b).start()

                @pl.when(c + _NBUF - 1 < nchunks)
                def _():
                    idx_descr(c + _NBUF - 1, nxt).wait()
                    w_descr(c + _NBUF - 1, nxt).wait()
                    for d in gather_descr(nxt):
                        d.start()

                @pl.when(c >= 2)
                def _():
                    out_descr(c - 2, ob).wait()

                compute(b, ob)
                out_descr(c, ob).start()

                @pl.when(c + _NBUF < nchunks)
                def _():
                    w_descr(c + _NBUF, b).start()
            return carry

        lax.fori_loop(0, nchunks // _NBUF, ring_body, 0)

        out_descr(0, 0).wait()
        out_descr(0, 1).wait()

    return polar_sc, idx_all, w_all


def kernel(img):
    B, H, W, C = img.shape
    polar_sc, idx_all, w_all = _build(B, H, W, C)
    table = img.reshape(B * H * W, C)
    return polar_sc(table, jnp.asarray(idx_all), jnp.asarray(w_all))
